# baseline (speedup 1.0000x reference)
import functools

import jax
import jax.numpy as jnp
from jax import lax
from jax.experimental import pallas as pl
from jax.experimental.pallas import tpu as pltpu
from jax.experimental.pallas import tpu_sc as plsc

_TOP_K = 16
_N_ITER = 4
_EARLY_STOP = 2
_REDIST_W = 0.9
_ROWS = 8


def _search_body(p_ref, noise_ref, samples_ref, svals_ref, sum_ref):
    p = p_ref[...]
    n_rows, v = p.shape

    cw_lanes = 2048
    nfull = v // cw_lanes
    vmain = nfull * cw_lanes
    tail_n = v - vmain
    nc = nfull + 1
    nhi = nfull // 8
    p3 = p[:, :vmain].reshape(n_rows, nfull, cw_lanes)
    p4 = p[:, :vmain].reshape(n_rows, nhi, 8, cw_lanes)
    tail_vec = jnp.concatenate(
        [p[:, vmain:],
         jnp.full((n_rows, cw_lanes - tail_n), -jnp.inf, jnp.float32)],
        axis=1)
    iota_nc = lax.broadcasted_iota(jnp.int32, (n_rows, nc), 1)
    iota_cw = lax.broadcasted_iota(jnp.int32, (n_rows, cw_lanes), 1)

    sum_ref[...] = (jnp.sum(jnp.sum(p3, axis=1), axis=1, keepdims=True)
                    + jnp.sum(p[:, vmain:], axis=1, keepdims=True))
    del p

    cm = jnp.concatenate(
        [jnp.max(p3, axis=2),
         jnp.max(tail_vec, axis=1, keepdims=True)], axis=1)

    excls, vals_l, idx_l = [], [], []
    for _ in range(_TOP_K):
        m = jnp.max(cm, axis=1, keepdims=True)
        cw = jnp.min(jnp.where(cm == m, iota_nc, nc), axis=1, keepdims=True)
        cwc = jnp.minimum(cw, nfull - 1)
        lo4 = jnp.broadcast_to((cwc % 8)[:, :, None, None],
                               (n_rows, nhi, 1, cw_lanes))
        l1 = jnp.take_along_axis(p4, lo4, axis=2)[:, :, 0, :]
        hi3 = jnp.broadcast_to((cwc // 8)[:, :, None], (n_rows, 1, cw_lanes))
        comp = jnp.take_along_axis(l1, hi3, axis=1)[:, 0, :]
        comp = jnp.where(cw == nfull, tail_vec, comp)
        glob = cw * cw_lanes + iota_cw
        for e in excls:
            comp = jnp.where(glob == e, -jnp.inf, comp)
        lw = jnp.min(jnp.where(comp == m, iota_cw, cw_lanes), axis=1,
                     keepdims=True)
        gidx = cw * cw_lanes + lw
        vals_l.append(m)
        idx_l.append(gidx)
        excls.append(gidx)
        comp2 = jnp.where(iota_cw == lw, -jnp.inf, comp)
        newm = jnp.max(comp2, axis=1, keepdims=True)
        cm = jnp.where(iota_nc == cw, newm, cm)

    vals = jnp.concatenate(vals_l, axis=1)
    idx = jnp.concatenate(idx_l, axis=1)
    iota_k = lax.broadcasted_iota(jnp.int32, (n_rows, _TOP_K), 1)

    logp = jnp.log(vals)
    rows = pl.program_id(0) * n_rows + lax.broadcasted_iota(
        jnp.int32, (n_rows, 1), 0)
    is_l0 = (rows % 32) == 0
    map_idx = idx[:, 0:1]
    map_val = vals[:, 0:1]
    for it in range(_N_ITER):
        z = logp + noise_ref[it]
        zm = jnp.max(z, axis=1, keepdims=True)
        c = jnp.min(jnp.where(z == zm, iota_k, _TOP_K), axis=1, keepdims=True)
        sel = iota_k == c
        s_it = jnp.sum(jnp.where(sel, idx, 0), axis=1, keepdims=True)
        v_it = jnp.sum(jnp.where(sel, vals, 0.0), axis=1, keepdims=True)
        s_it = jnp.where(is_l0, map_idx, s_it)
        v_it = jnp.where(is_l0, map_val, v_it)
        samples_ref[0, it, :] = s_it[:, 0]
        svals_ref[0, it, :] = v_it[:, 0]


def _gather_rows(table, idx):
    v, d = table.shape
    bn = idx.shape[0]
    info = plsc.get_sparse_core_info()
    nc, ns = info.num_cores, info.num_subcores
    nw = nc * ns
    b_per_w = bn // nw
    mesh = plsc.VectorSubcoreMesh(core_axis_name="c", subcore_axis_name="s")

    @functools.partial(
        pl.kernel,
        mesh=mesh,
        out_type=jax.ShapeDtypeStruct((bn, d), jnp.float32),
        scratch_types=[
            pltpu.VMEM((b_per_w,), jnp.int32),
            pltpu.VMEM((b_per_w, d), jnp.float32),
            pltpu.SemaphoreType.DMA,
        ],
    )
    def k(table_hbm, idx_hbm, out_hbm, idx_v, rows_v, sem):
        wid = lax.axis_index("s") * nc + lax.axis_index("c")
        base = wid * b_per_w
        pltpu.sync_copy(idx_hbm.at[pl.ds(base, b_per_w)], idx_v)
        pltpu.async_copy(table_hbm.at[idx_v], rows_v, sem).wait()
        pltpu.sync_copy(rows_v, out_hbm.at[pl.ds(base, b_per_w)])

    return k(table, idx)


def _score_body(scale_ref, embs_ref, samples_ref, svals_ref, sums_ref,
                ms_ref, bc_ref, sa_ref, sb_ref, k_scratch):
    nb = samples_ref.shape[1]
    ll = samples_ref.shape[2]
    nm_mat = _N_ITER * nb
    sc2 = scale_ref[0, 0] * scale_ref[0, 0]
    embs = embs_ref[...]
    for m in range(nm_mat):
        e = embs[m * ll:(m + 1) * ll, :]
        gram = lax.dot_general(e, e, (((1,), (1,)), ((), ())),
                               preferred_element_type=jnp.float32)
        k_scratch[m] = gram * sc2

    a0 = k_scratch[...]
    r_i = lax.broadcasted_iota(jnp.int32, (nm_mat, ll, ll), 1)
    c_i = lax.broadcasted_iota(jnp.int32, (nm_mat, ll, ll), 2)
    d_i = lax.broadcasted_iota(jnp.int32, (nm_mat, ll), 1)

    def ge(j, carry):
        a, acc = carry
        colv = jnp.sum(jnp.where(c_i == j, a, 0.0), axis=2)
        rowv = jnp.sum(jnp.where(r_i == j, a, 0.0), axis=1)
        piv = jnp.sum(jnp.where(d_i == j, colv, 0.0), axis=1)
        inv = 1.0 / piv
        a = a - colv[:, :, None] * rowv[:, None, :] * inv[:, None, None]
        acc = acc + jnp.log(jnp.maximum(piv, 1e-30))
        return a, acc

    _, acc = lax.fori_loop(0, ll, ge, (a0, jnp.zeros((nm_mat,), jnp.float32)))

    scores = [acc[it * nb:(it + 1) * nb] for it in range(_N_ITER)]
    max_score = scores[0]
    best_it = jnp.zeros((nb,), jnp.int32)
    active = jnp.int32(1)
    count = jnp.int32(0)
    for it in range(1, _N_ITER):
        maxd = max_score < scores[it]
        improved = jnp.any(maxd)
        count = jnp.where(active == 1,
                          jnp.where(improved, jnp.int32(0), count + 1), count)
        break_now = count >= _EARLY_STOP
        upd = maxd & (active == 1) & jnp.logical_not(break_now)
        max_score = jnp.where(upd, scores[it], max_score)
        best_it = jnp.where(upd, jnp.int32(it), best_it)
        active = jnp.where(break_now, jnp.int32(0), active)

    bc = samples_ref[0]
    pb = svals_ref[0]
    for it in range(1, _N_ITER):
        m_it = (best_it == it)[:, None]
        bc = jnp.where(m_it, samples_ref[it], bc)
        pb = jnp.where(m_it, svals_ref[it], pb)
    s = sums_ref[...]
    w = _REDIST_W
    nm = (1.0 - w) * s + (2.0 * w - 1.0) * pb
    ms_ref[0, :] = max_score
    bc_ref[...] = bc
    sa_ref[...] = (1.0 - w) / nm
    sb_ref[...] = w / nm


def _rescale_body(p_ref, bc_ref, sa_ref, sb_ref, o_ref):
    p = p_ref[...]
    iota_v = lax.broadcasted_iota(jnp.int32, p.shape, 1)
    o_ref[...] = jnp.where(iota_v == bc_ref[...], p * sb_ref[...],
                           p * sa_ref[...])


def kernel(probas, h_d, mask, emb_weight, embed_scale):
    b, l, v = probas.shape
    r = b * l
    d = emb_weight.shape[1]
    p2 = probas.reshape(r, v)

    base_key = jax.random.key(42)
    noise = jnp.stack([
        jax.random.gumbel(jax.random.fold_in(base_key, it), (b, l, _TOP_K),
                          jnp.float32) for it in range(_N_ITER)
    ]).reshape(_N_ITER, r, _TOP_K)

    samples, svals, sums = pl.pallas_call(
        _search_body,
        grid=(r // _ROWS,),
        in_specs=[
            pl.BlockSpec((_ROWS, v), lambda i: (i, 0)),
            pl.BlockSpec((_N_ITER, _ROWS, _TOP_K), lambda i: (0, i, 0)),
        ],
        out_specs=[
            pl.BlockSpec((1, _N_ITER, _ROWS), lambda i: (i, 0, 0)),
            pl.BlockSpec((1, _N_ITER, _ROWS), lambda i: (i, 0, 0)),
            pl.BlockSpec((_ROWS, 1), lambda i: (i, 0)),
        ],
        out_shape=[
            jax.ShapeDtypeStruct((r // _ROWS, _N_ITER, _ROWS), jnp.int32),
            jax.ShapeDtypeStruct((r // _ROWS, _N_ITER, _ROWS), jnp.float32),
            jax.ShapeDtypeStruct((r, 1), jnp.float32),
        ],
    )(p2, noise)
    samples = samples.transpose(1, 0, 2).reshape(_N_ITER, r)
    svals = svals.transpose(1, 0, 2).reshape(_N_ITER, r)

    embs = _gather_rows(emb_weight, samples.reshape(-1))

    ms, bc, sa, sb = pl.pallas_call(
        _score_body,
        in_specs=[
            pl.BlockSpec(memory_space=pltpu.SMEM),
            pl.BlockSpec((_N_ITER * r, d), lambda: (0, 0)),
            pl.BlockSpec((_N_ITER, b, l), lambda: (0, 0, 0)),
            pl.BlockSpec((_N_ITER, b, l), lambda: (0, 0, 0)),
            pl.BlockSpec((b, l), lambda: (0, 0)),
        ],
        out_specs=[
            pl.BlockSpec((1, b), lambda: (0, 0)),
            pl.BlockSpec((b, l), lambda: (0, 0)),
            pl.BlockSpec((b, l), lambda: (0, 0)),
            pl.BlockSpec((b, l), lambda: (0, 0)),
        ],
        out_shape=[
            jax.ShapeDtypeStruct((1, b), jnp.float32),
            jax.ShapeDtypeStruct((b, l), jnp.int32),
            jax.ShapeDtypeStruct((b, l), jnp.float32),
            jax.ShapeDtypeStruct((b, l), jnp.float32),
        ],
        scratch_shapes=[pltpu.VMEM((_N_ITER * b, l, l), jnp.float32)],
    )(embed_scale.reshape(1, 1), embs, samples.reshape(_N_ITER, b, l),
      svals.reshape(_N_ITER, b, l), sums.reshape(b, l))

    new_probas = pl.pallas_call(
        _rescale_body,
        grid=(r // _ROWS,),
        in_specs=[
            pl.BlockSpec((_ROWS, v), lambda i: (i, 0)),
            pl.BlockSpec((_ROWS, 1), lambda i: (i, 0)),
            pl.BlockSpec((_ROWS, 1), lambda i: (i, 0)),
            pl.BlockSpec((_ROWS, 1), lambda i: (i, 0)),
        ],
        out_specs=pl.BlockSpec((_ROWS, v), lambda i: (i, 0)),
        out_shape=jax.ShapeDtypeStruct((r, v), jnp.float32),
    )(p2, bc.reshape(r, 1), sa.reshape(r, 1), sb.reshape(r, 1))

    return new_probas.reshape(b, l, v), ms.reshape(b)


# speedup vs baseline: 1.0104x; 1.0104x over previous
import functools

import jax
import jax.numpy as jnp
from jax import lax
from jax.experimental import pallas as pl
from jax.experimental.pallas import tpu as pltpu
from jax.experimental.pallas import tpu_sc as plsc

_TOP_K = 16
_N_ITER = 4
_EARLY_STOP = 2
_REDIST_W = 0.9
_ROWS = 8


def _search_body(p_ref, noise_ref, samples_ref, svals_ref, sum_ref):
    p = p_ref[...]
    n_rows, v = p.shape

    cw_lanes = 1024
    nfull = v // cw_lanes
    vmain = nfull * cw_lanes
    tail_n = v - vmain
    nc = nfull + 1
    p3 = p[:, :vmain].reshape(n_rows, nfull, cw_lanes)
    tail_vec = jnp.concatenate(
        [p[:, vmain:],
         jnp.full((n_rows, cw_lanes - tail_n), -jnp.inf, jnp.float32)],
        axis=1)
    iota_nc = lax.broadcasted_iota(jnp.int32, (n_rows, nc), 1)
    iota_cw = lax.broadcasted_iota(jnp.int32, (n_rows, cw_lanes), 1)
    chunk3 = lax.broadcasted_iota(jnp.int32, (n_rows, nfull, cw_lanes), 1)

    sum_ref[...] = (jnp.sum(jnp.sum(p3, axis=1), axis=1, keepdims=True)
                    + jnp.sum(p[:, vmain:], axis=1, keepdims=True))
    del p

    cm = jnp.concatenate(
        [jnp.max(p3, axis=2),
         jnp.max(tail_vec, axis=1, keepdims=True)], axis=1)

    excls, vals_l, idx_l = [], [], []
    for _ in range(_TOP_K):
        m = jnp.max(cm, axis=1, keepdims=True)
        cw = jnp.min(jnp.where(cm == m, iota_nc, nc), axis=1, keepdims=True)
        mask3 = chunk3 == cw[:, :, None]
        comp = jnp.sum(jnp.where(mask3, p3, 0.0), axis=1)
        comp = jnp.where(cw == nfull, comp + tail_vec, comp)
        glob = cw * cw_lanes + iota_cw
        for e in excls:
            comp = jnp.where(glob == e, -jnp.inf, comp)
        lw = jnp.min(jnp.where(comp == m, iota_cw, cw_lanes), axis=1,
                     keepdims=True)
        gidx = cw * cw_lanes + lw
        vals_l.append(m)
        idx_l.append(gidx)
        excls.append(gidx)
        comp2 = jnp.where(iota_cw == lw, -jnp.inf, comp)
        newm = jnp.max(comp2, axis=1, keepdims=True)
        cm = jnp.where(iota_nc == cw, newm, cm)

    vals = jnp.concatenate(vals_l, axis=1)
    idx = jnp.concatenate(idx_l, axis=1)
    iota_k = lax.broadcasted_iota(jnp.int32, (n_rows, _TOP_K), 1)

    logp = jnp.log(vals)
    rows = pl.program_id(0) * n_rows + lax.broadcasted_iota(
        jnp.int32, (n_rows, 1), 0)
    is_l0 = (rows % 32) == 0
    map_idx = idx[:, 0:1]
    map_val = vals[:, 0:1]
    for it in range(_N_ITER):
        z = logp + noise_ref[it]
        zm = jnp.max(z, axis=1, keepdims=True)
        c = jnp.min(jnp.where(z == zm, iota_k, _TOP_K), axis=1, keepdims=True)
        sel = iota_k == c
        s_it = jnp.sum(jnp.where(sel, idx, 0), axis=1, keepdims=True)
        v_it = jnp.sum(jnp.where(sel, vals, 0.0), axis=1, keepdims=True)
        s_it = jnp.where(is_l0, map_idx, s_it)
        v_it = jnp.where(is_l0, map_val, v_it)
        samples_ref[0, it, :] = s_it[:, 0]
        svals_ref[0, it, :] = v_it[:, 0]


def _gather_rows(table, idx):
    v, d = table.shape
    bn = idx.shape[0]
    info = plsc.get_sparse_core_info()
    nc, ns = info.num_cores, info.num_subcores
    nw = nc * ns
    b_per_w = bn // nw
    mesh = plsc.VectorSubcoreMesh(core_axis_name="c", subcore_axis_name="s")

    @functools.partial(
        pl.kernel,
        mesh=mesh,
        out_type=jax.ShapeDtypeStruct((bn, d), jnp.float32),
        scratch_types=[
            pltpu.VMEM((b_per_w,), jnp.int32),
            pltpu.VMEM((b_per_w, d), jnp.float32),
            pltpu.SemaphoreType.DMA,
        ],
    )
    def k(table_hbm, idx_hbm, out_hbm, idx_v, rows_v, sem):
        wid = lax.axis_index("s") * nc + lax.axis_index("c")
        base = wid * b_per_w
        pltpu.sync_copy(idx_hbm.at[pl.ds(base, b_per_w)], idx_v)
        pltpu.async_copy(table_hbm.at[idx_v], rows_v, sem).wait()
        pltpu.sync_copy(rows_v, out_hbm.at[pl.ds(base, b_per_w)])

    return k(table, idx)


def _score_body(scale_ref, embs_ref, samples_ref, svals_ref, sums_ref,
                ms_ref, bc_ref, sa_ref, sb_ref, k_scratch):
    nb = samples_ref.shape[1]
    ll = samples_ref.shape[2]
    nm_mat = _N_ITER * nb
    sc2 = scale_ref[0, 0] * scale_ref[0, 0]
    embs = embs_ref[...]
    for m in range(nm_mat):
        e = embs[m * ll:(m + 1) * ll, :]
        gram = lax.dot_general(e, e, (((1,), (1,)), ((), ())),
                               preferred_element_type=jnp.float32)
        k_scratch[m] = gram * sc2

    a0 = k_scratch[...]
    r_i = lax.broadcasted_iota(jnp.int32, (nm_mat, ll, ll), 1)
    c_i = lax.broadcasted_iota(jnp.int32, (nm_mat, ll, ll), 2)
    d_i = lax.broadcasted_iota(jnp.int32, (nm_mat, ll), 1)

    def ge(j, carry):
        a, acc = carry
        colv = jnp.sum(jnp.where(c_i == j, a, 0.0), axis=2)
        rowv = jnp.sum(jnp.where(r_i == j, a, 0.0), axis=1)
        piv = jnp.sum(jnp.where(d_i == j, colv, 0.0), axis=1)
        inv = 1.0 / piv
        a = a - colv[:, :, None] * rowv[:, None, :] * inv[:, None, None]
        acc = acc + jnp.log(jnp.maximum(piv, 1e-30))
        return a, acc

    _, acc = lax.fori_loop(0, ll, ge, (a0, jnp.zeros((nm_mat,), jnp.float32)))

    scores = [acc[it * nb:(it + 1) * nb] for it in range(_N_ITER)]
    max_score = scores[0]
    best_it = jnp.zeros((nb,), jnp.int32)
    active = jnp.int32(1)
    count = jnp.int32(0)
    for it in range(1, _N_ITER):
        maxd = max_score < scores[it]
        improved = jnp.any(maxd)
        count = jnp.where(active == 1,
                          jnp.where(improved, jnp.int32(0), count + 1), count)
        break_now = count >= _EARLY_STOP
        upd = maxd & (active == 1) & jnp.logical_not(break_now)
        max_score = jnp.where(upd, scores[it], max_score)
        best_it = jnp.where(upd, jnp.int32(it), best_it)
        active = jnp.where(break_now, jnp.int32(0), active)

    bc = samples_ref[0]
    pb = svals_ref[0]
    for it in range(1, _N_ITER):
        m_it = (best_it == it)[:, None]
        bc = jnp.where(m_it, samples_ref[it], bc)
        pb = jnp.where(m_it, svals_ref[it], pb)
    s = sums_ref[...]
    w = _REDIST_W
    nm = (1.0 - w) * s + (2.0 * w - 1.0) * pb
    ms_ref[0, :] = max_score
    bc_ref[...] = bc
    sa_ref[...] = (1.0 - w) / nm
    sb_ref[...] = w / nm


def _rescale_body(p_ref, bc_ref, sa_ref, sb_ref, o_ref):
    p = p_ref[...]
    iota_v = lax.broadcasted_iota(jnp.int32, p.shape, 1)
    o_ref[...] = jnp.where(iota_v == bc_ref[...], p * sb_ref[...],
                           p * sa_ref[...])


def kernel(probas, h_d, mask, emb_weight, embed_scale):
    b, l, v = probas.shape
    r = b * l
    d = emb_weight.shape[1]
    p2 = probas.reshape(r, v)

    base_key = jax.random.key(42)
    noise = jnp.stack([
        jax.random.gumbel(jax.random.fold_in(base_key, it), (b, l, _TOP_K),
                          jnp.float32) for it in range(_N_ITER)
    ]).reshape(_N_ITER, r, _TOP_K)

    samples, svals, sums = pl.pallas_call(
        _search_body,
        grid=(r // _ROWS,),
        in_specs=[
            pl.BlockSpec((_ROWS, v), lambda i: (i, 0)),
            pl.BlockSpec((_N_ITER, _ROWS, _TOP_K), lambda i: (0, i, 0)),
        ],
        out_specs=[
            pl.BlockSpec((1, _N_ITER, _ROWS), lambda i: (i, 0, 0)),
            pl.BlockSpec((1, _N_ITER, _ROWS), lambda i: (i, 0, 0)),
            pl.BlockSpec((_ROWS, 1), lambda i: (i, 0)),
        ],
        out_shape=[
            jax.ShapeDtypeStruct((r // _ROWS, _N_ITER, _ROWS), jnp.int32),
            jax.ShapeDtypeStruct((r // _ROWS, _N_ITER, _ROWS), jnp.float32),
            jax.ShapeDtypeStruct((r, 1), jnp.float32),
        ],
    )(p2, noise)
    samples = samples.transpose(1, 0, 2).reshape(_N_ITER, r)
    svals = svals.transpose(1, 0, 2).reshape(_N_ITER, r)

    embs = _gather_rows(emb_weight, samples.reshape(-1))

    ms, bc, sa, sb = pl.pallas_call(
        _score_body,
        in_specs=[
            pl.BlockSpec(memory_space=pltpu.SMEM),
            pl.BlockSpec((_N_ITER * r, d), lambda: (0, 0)),
            pl.BlockSpec((_N_ITER, b, l), lambda: (0, 0, 0)),
            pl.BlockSpec((_N_ITER, b, l), lambda: (0, 0, 0)),
            pl.BlockSpec((b, l), lambda: (0, 0)),
        ],
        out_specs=[
            pl.BlockSpec((1, b), lambda: (0, 0)),
            pl.BlockSpec((b, l), lambda: (0, 0)),
            pl.BlockSpec((b, l), lambda: (0, 0)),
            pl.BlockSpec((b, l), lambda: (0, 0)),
        ],
        out_shape=[
            jax.ShapeDtypeStruct((1, b), jnp.float32),
            jax.ShapeDtypeStruct((b, l), jnp.int32),
            jax.ShapeDtypeStruct((b, l), jnp.float32),
            jax.ShapeDtypeStruct((b, l), jnp.float32),
        ],
        scratch_shapes=[pltpu.VMEM((_N_ITER * b, l, l), jnp.float32)],
    )(embed_scale.reshape(1, 1), embs, samples.reshape(_N_ITER, b, l),
      svals.reshape(_N_ITER, b, l), sums.reshape(b, l))

    new_probas = pl.pallas_call(
        _rescale_body,
        grid=(r // _ROWS,),
        in_specs=[
            pl.BlockSpec((_ROWS, v), lambda i: (i, 0)),
            pl.BlockSpec((_ROWS, 1), lambda i: (i, 0)),
            pl.BlockSpec((_ROWS, 1), lambda i: (i, 0)),
            pl.BlockSpec((_ROWS, 1), lambda i: (i, 0)),
        ],
        out_specs=pl.BlockSpec((_ROWS, v), lambda i: (i, 0)),
        out_shape=jax.ShapeDtypeStruct((r, v), jnp.float32),
    )(p2, bc.reshape(r, 1), sa.reshape(r, 1), sb.reshape(r, 1))

    return new_probas.reshape(b, l, v), ms.reshape(b)
